# baseline (speedup 1.0000x reference)
import functools
import math

import jax
import jax.numpy as jnp
from jax import lax
from jax.experimental import pallas as pl
from jax.experimental.pallas import tpu as pltpu
from jax.experimental.pallas import tpu_sc as plsc

D_MODEL = 64
SCALE_F = float(math.sqrt(D_MODEL))
CLEN = 128
LANES = 16
VB = 160

_info = plsc.get_sparse_core_info()
_NC = _info.num_cores
_NS = _info.num_subcores
_NW = _NC * _NS

_params = pltpu.CompilerParams(
    use_tc_tiling_on_sc=True, needs_layout_passes=False
)


@functools.lru_cache(maxsize=None)
def _make_sc_compact(vocab: int):
    n_blk = vocab // VB
    n_it = 2 * (-(-n_blk // (2 * _NW)))
    mesh = plsc.VectorSubcoreMesh(core_axis_name="c", subcore_axis_name="s")

    @functools.partial(
        pl.kernel,
        mesh=mesh,
        out_type=jax.ShapeDtypeStruct((vocab // 2, 2 * D_MODEL), jnp.float32),
        scratch_types=[
            pltpu.VMEM((2, VB, D_MODEL), jnp.float32),
            pltpu.VMEM((2, VB // 2, 2 * D_MODEL), jnp.float32),
            pltpu.SemaphoreType.DMA((2,)),
            pltpu.SemaphoreType.DMA((2,)),
        ],
        compiler_params=_params,
    )
    def k(tab_hbm, out_hbm, ibuf, obuf, isem, osem):
        c = lax.axis_index("c")
        s = lax.axis_index("s")
        wid = s * _NC + c

        def kk(n):
            return lax.rem(wid + _NW * n, n_blk)

        for b in range(2):
            pltpu.async_copy(
                tab_hbm.at[pl.ds(kk(b) * VB, VB)], ibuf.at[b], isem.at[b]
            )

        def body(n2, carry):
            for b in range(2):
                n = n2 * 2 + b
                pltpu.make_async_copy(
                    tab_hbm.at[pl.ds(0, VB)], ibuf.at[b], isem.at[b]
                ).wait()

                @pl.when(n >= 2)
                def _wait_store(b=b):
                    pltpu.make_async_copy(
                        obuf.at[b], out_hbm.at[pl.ds(0, VB // 2)], osem.at[b]
                    ).wait()

                def pack(p, cc, b=b):
                    for h in range(2):
                        for c4 in range(D_MODEL // LANES):
                            sl = pl.ds(c4 * LANES, LANES)
                            dsl = pl.ds(h * D_MODEL + c4 * LANES, LANES)
                            obuf[b, p, dsl] = ibuf[b, 2 * p + h, sl]
                    return cc

                lax.fori_loop(0, VB // 2, pack, 0)
                pltpu.make_async_copy(
                    obuf.at[b],
                    out_hbm.at[pl.ds(kk(n) * (VB // 2), VB // 2)],
                    osem.at[b],
                ).start()

                @pl.when(n + 2 < n_it)
                def _next(n=n, b=b):
                    pltpu.async_copy(
                        tab_hbm.at[pl.ds(kk(n + 2) * VB, VB)],
                        ibuf.at[b], isem.at[b],
                    )
            return carry

        lax.fori_loop(0, n_it // 2, body, 0)

        for b in range(2):
            pltpu.make_async_copy(
                obuf.at[b], out_hbm.at[pl.ds(0, VB // 2)], osem.at[b]
            ).wait()

    return k


@functools.lru_cache(maxsize=None)
def _make_sc_gather(n_i: int, n_t: int, vocab_pairs: int):
    cb_total = n_i // CLEN
    cb_per_w = cb_total // _NW
    n_chunks = cb_per_w * n_t
    mesh = plsc.VectorSubcoreMesh(core_axis_name="c", subcore_axis_name="s")

    @functools.partial(
        pl.kernel,
        mesh=mesh,
        out_type=jax.ShapeDtypeStruct((n_t, D_MODEL, n_i), jnp.float32),
        scratch_types=[
            pltpu.VMEM((n_chunks, CLEN), jnp.int32),
            pltpu.VMEM((4, CLEN), jnp.int32),
            pltpu.VMEM((4, CLEN, 128), jnp.float32),
            pltpu.VMEM((2, D_MODEL, CLEN), jnp.float32),
            pltpu.SemaphoreType.DMA((4,)),
            pltpu.SemaphoreType.DMA((2,)),
        ],
        compiler_params=_params,
    )
    def k(table_hbm, idx_hbm, out_hbm, idx_v, pstg, gbuf, stg, gsem, ssem):
        c = lax.axis_index("c")
        s = lax.axis_index("s")
        wid = s * _NC + c
        pltpu.sync_copy(idx_hbm.at[wid], idx_v)
        c0 = wid * cb_per_w
        iot = lax.iota(jnp.int32, LANES)

        def derive_pairs(g, b):
            for kb in range(CLEN // LANES):
                sl = pl.ds(kb * LANES, LANES)
                pstg[b, sl] = idx_v[g, sl] >> 1

        for b in range(4):
            derive_pairs(b, b)
            pltpu.async_copy(
                table_hbm.at[pstg.at[b]], gbuf.at[b], gsem.at[b]
            )

        def chunk_body(g4, carry):
            for b in range(4):
                g = g4 * 4 + b
                sb = b % 2
                pltpu.make_async_copy(
                    table_hbm.at[pstg.at[b]], gbuf.at[b], gsem.at[b]
                ).wait()

                @pl.when(g >= 2)
                def _wait_prev_store(sb=sb):
                    pltpu.make_async_copy(
                        stg.at[sb], out_hbm.at[0, :, pl.ds(0, CLEN)],
                        ssem.at[sb],
                    ).wait()

                def blk_body(rb, cc, b=b, g=g, sb=sb):
                    r0 = rb * LANES
                    rows = iot + r0
                    h16 = (idx_v[g, pl.ds(r0, LANES)] & 1) * D_MODEL
                    for jb in range(D_MODEL // LANES):
                        j0 = jb * LANES
                        for d in range(LANES):
                            jrot = ((iot + d) & (LANES - 1)) + j0
                            v = plsc.load_gather(
                                gbuf.at[b], [rows, h16 + jrot]
                            )
                            plsc.store_scatter(
                                stg.at[sb], [jrot, rows], v * SCALE_F
                            )
                    return cc

                lax.fori_loop(0, CLEN // LANES, blk_body, 0)

                t = g // cb_per_w
                cl = lax.rem(g, cb_per_w)
                i0 = (c0 + cl) * CLEN
                pltpu.make_async_copy(
                    stg.at[sb], out_hbm.at[t, :, pl.ds(i0, CLEN)],
                    ssem.at[sb],
                ).start()

                @pl.when(g + 4 < n_chunks)
                def _next_gather(g=g, b=b):
                    derive_pairs(g + 4, b)
                    pltpu.async_copy(
                        table_hbm.at[pstg.at[b]], gbuf.at[b], gsem.at[b]
                    )
            return carry

        lax.fori_loop(0, n_chunks // 4, chunk_body, 0)

        for sb in range(2):
            pltpu.make_async_copy(
                stg.at[sb], out_hbm.at[0, :, pl.ds(0, CLEN)], ssem.at[sb]
            ).wait()

    return k


def kernel(x, table):
    n_b, n_t = x.shape
    vocab = table.shape[0]
    n_i = n_b
    assert n_i % (_NW * CLEN) == 0 and vocab % (2 * _NW) == 0
    cb_per_w = n_i // (_NW * CLEN)

    xi = x.astype(jnp.int32)
    xw = (
        xi.T.reshape(n_t, _NW, cb_per_w, CLEN)
        .transpose(1, 0, 2, 3)
        .reshape(_NW, n_t * cb_per_w, CLEN)
    )
    packed = _make_sc_compact(vocab)(table)
    out = _make_sc_gather(n_i, n_t, vocab // 2)(packed, xw)
    return jnp.transpose(out, (2, 0, 1))


# speedup vs baseline: 1.1940x; 1.1940x over previous
import functools
import math

import jax
import jax.numpy as jnp
from jax import lax
from jax.experimental import pallas as pl
from jax.experimental.pallas import tpu as pltpu
from jax.experimental.pallas import tpu_sc as plsc

D_MODEL = 64
SCALE_F = float(math.sqrt(D_MODEL))
CLEN = 128
LANES = 16

_info = plsc.get_sparse_core_info()
_NC = _info.num_cores
_NS = _info.num_subcores
_NW = _NC * _NS


@functools.lru_cache(maxsize=None)
def _make_sc_gather(n_i: int, n_t: int, vocab_pairs: int):
    cb_total = n_i // CLEN
    cb_per_w = cb_total // _NW
    n_chunks = cb_per_w * n_t
    mesh = plsc.VectorSubcoreMesh(core_axis_name="c", subcore_axis_name="s")

    @functools.partial(
        pl.kernel,
        mesh=mesh,
        out_type=jax.ShapeDtypeStruct((n_t, D_MODEL, n_i), jnp.float32),
        scratch_types=[
            pltpu.VMEM((n_chunks, CLEN), jnp.int32),
            pltpu.VMEM((n_chunks, CLEN), jnp.int32),
            pltpu.VMEM((2, CLEN, 128), jnp.float32),
            pltpu.VMEM((2, D_MODEL, CLEN), jnp.float32),
            pltpu.SemaphoreType.DMA((2,)),
            pltpu.SemaphoreType.DMA((2,)),
        ],
        compiler_params=pltpu.CompilerParams(
            use_tc_tiling_on_sc=True, needs_layout_passes=False
        ),
    )
    def k(table_hbm, pair_hbm, half_hbm, out_hbm, pair_v, half_v, gbuf,
          stg, gsem, ssem):
        c = lax.axis_index("c")
        s = lax.axis_index("s")
        wid = s * _NC + c
        pltpu.sync_copy(pair_hbm.at[wid], pair_v)
        pltpu.sync_copy(half_hbm.at[wid], half_v)
        c0 = wid * cb_per_w
        iot = lax.iota(jnp.int32, LANES)

        for b in range(2):
            pltpu.async_copy(
                table_hbm.at[pair_v.at[b]], gbuf.at[b], gsem.at[b]
            )

        def chunk_body(g2, carry):
            for b in range(2):
                g = g2 * 2 + b
                pltpu.make_async_copy(
                    table_hbm.at[pair_v.at[g]], gbuf.at[b], gsem.at[b]
                ).wait()

                @pl.when(g >= 2)
                def _wait_prev_store(b=b):
                    pltpu.make_async_copy(
                        stg.at[b], out_hbm.at[0, :, pl.ds(0, CLEN)],
                        ssem.at[b],
                    ).wait()

                def blk_body(rb, cc, b=b, g=g):
                    r0 = rb * LANES
                    rows = iot + r0
                    h16 = half_v[g, pl.ds(r0, LANES)]
                    for jb in range(D_MODEL // LANES):
                        j0 = jb * LANES
                        for d in range(LANES):
                            jrot = ((iot + d) & (LANES - 1)) + j0
                            v = plsc.load_gather(
                                gbuf.at[b], [rows, h16 + jrot]
                            )
                            plsc.store_scatter(
                                stg.at[b], [jrot, rows], v * SCALE_F
                            )
                    return cc

                lax.fori_loop(0, CLEN // LANES, blk_body, 0)

                t = g // cb_per_w
                cl = lax.rem(g, cb_per_w)
                i0 = (c0 + cl) * CLEN
                pltpu.make_async_copy(
                    stg.at[b], out_hbm.at[t, :, pl.ds(i0, CLEN)], ssem.at[b]
                ).start()

                @pl.when(g + 2 < n_chunks)
                def _next_gather(g=g, b=b):
                    pltpu.async_copy(
                        table_hbm.at[pair_v.at[g + 2]], gbuf.at[b],
                        gsem.at[b],
                    )
            return carry

        lax.fori_loop(0, n_chunks // 2, chunk_body, 0)

        for b in range(2):
            pltpu.make_async_copy(
                stg.at[b], out_hbm.at[0, :, pl.ds(0, CLEN)], ssem.at[b]
            ).wait()

    return k


def kernel(x, table):
    n_b, n_t = x.shape
    vocab = table.shape[0]
    n_i = n_b
    assert n_i % (_NW * CLEN) == 0 and vocab % 2 == 0
    cb_per_w = n_i // (_NW * CLEN)

    xi = x.astype(jnp.int32)
    xw = (
        xi.T.reshape(n_t, _NW, cb_per_w, CLEN)
        .transpose(1, 0, 2, 3)
        .reshape(_NW, n_t * cb_per_w, CLEN)
    )
    pairs = xw >> 1
    halves = (xw & 1) * D_MODEL
    table2 = table.reshape(vocab // 2, 2 * D_MODEL)
    out = _make_sc_gather(n_i, n_t, vocab // 2)(table2, pairs, halves)
    return jnp.transpose(out, (2, 0, 1))
